# baseline (speedup 1.0000x reference)
import functools

import jax
import jax.numpy as jnp
from jax import lax
from jax.experimental import pallas as pl
from jax.experimental.pallas import tpu as pltpu
from jax.experimental.pallas import tpu_sc as plsc

_C = 128
_NS = 16
_NC = 2
_H = 128
_BN = 1000


_NB = 2


@functools.lru_cache(maxsize=None)
def _make_segsum(n_nodes, n_edges, split):
    per_tile = n_edges // (_NS * (_NC if split else 1))
    n_chunks = per_tile // _C
    assert per_tile % _C == 0 and n_chunks % _NB == 0
    _BR = 1000
    nz_tiles = n_nodes // _BR

    mesh = plsc.VectorSubcoreMesh(core_axis_name="c", subcore_axis_name="s")

    scratch = (
        [pltpu.VMEM((per_tile,), jnp.int32)]
        + [pltpu.VMEM((_C, _H), jnp.float32) for _ in range(_NB)]
        + [pltpu.VMEM((_C,), jnp.int32) for _ in range(_NB)]
        + [pltpu.VMEM((_C,), jnp.float32) for _ in range(_NB)]
        + [pltpu.VMEM_SHARED((n_nodes, _H), jnp.float32)]
        + [pltpu.SemaphoreType.DMA] * (3 * _NB + 1)
    )

    @functools.partial(
        pl.kernel,
        mesh=mesh,
        out_type=jax.ShapeDtypeStruct((_NC, n_nodes, _H), jnp.float32),
        scratch_types=scratch,
    )
    def segsum(tab_a, tab_b, srcv, dstv, wv, out, src_all, *bufs):
        rows = bufs[:_NB]
        dst_v = bufs[_NB:2 * _NB]
        w_v = bufs[2 * _NB:3 * _NB]
        acc = bufs[3 * _NB]
        gsem = bufs[3 * _NB + 1:4 * _NB + 1]
        ssem = bufs[4 * _NB + 1:5 * _NB + 1]
        dsem = bufs[5 * _NB + 1:6 * _NB + 1]
        isem = bufs[6 * _NB + 1]

        c = lax.axis_index("c")
        s = lax.axis_index("s")

        if split:
            base_e = (c * _NS + s) * per_tile
        else:
            base_e = s * per_tile

        cp_s = pltpu.async_copy(srcv.at[pl.ds(base_e, per_tile)], src_all, isem)

        def zero_body(i, _):
            for j in range(_H // 16):
                rows[0][i, pl.ds(j * 16, 16)] = jnp.zeros((16,), jnp.float32)
            return 0

        lax.fori_loop(0, _C, zero_body, 0)
        row0 = s * _BR

        @pl.when(s < nz_tiles)
        def _():
            off = 0
            while off < _BR:
                nn = min(_C, _BR - off)
                pltpu.sync_copy(rows[0].at[pl.ds(0, nn)], acc.at[pl.ds(row0 + off, nn)])
                off += nn

        cp_s.wait()

        _NSUB = 16
        _SCH = _C // _NSUB

        def _gather_parts(tab, k, j):
            return [pltpu.make_async_copy(
                        tab.at[src_all.at[pl.ds(k * _C + i * _SCH, _SCH)]],
                        rows[j].at[pl.ds(i * _SCH, _SCH)], gsem[j])
                    for i in range(_NSUB)]

        def gather_start(k, j):
            if split:
                for cp in _gather_parts(tab_a, k, j):
                    cp.start()
            else:
                @pl.when(c == 0)
                def _():
                    for cp in _gather_parts(tab_a, k, j):
                        cp.start()

                @pl.when(c == 1)
                def _():
                    for cp in _gather_parts(tab_b, k, j):
                        cp.start()

        def gather_wait(k, j):
            for cp in _gather_parts(tab_a, k, j):
                cp.wait()

        def dstcp(k, j):
            return pltpu.make_async_copy(
                dstv.at[pl.ds(base_e + k * _C, _C)], dst_v[j], dsem[j])

        def wcp(k, j):
            return pltpu.make_async_copy(
                wv.at[pl.ds(base_e + k * _C, _C)], w_v[j], dsem[j])

        def scatter(j):
            return pltpu.make_async_copy(rows[j], acc.at[dst_v[j]], ssem[j])

        for j in range(_NB - 1):
            gather_start(j, j)
            dstcp(j, j).start()
            wcp(j, j).start()

        plsc.subcore_barrier()

        def body(kk, _):
            for j in range(_NB):
                k = kk * _NB + j
                gather_wait(k, j)

                dstcp(k, j).wait()
                wcp(k, j).wait()

                pb = (j + _NB - 1) % _NB
                pf = k + _NB - 1
                prev = k - 1

                @pl.when(prev >= 0)
                def _():
                    scatter(pb).wait()

                @pl.when(pf < n_chunks)
                def _():
                    gather_start(pf, pb)
                    dstcp(pf, pb).start()
                    wcp(pf, pb).start()

                def mul_body(g, _):
                    wvec = w_v[j][pl.ds(g * 16, 16)]
                    for t in range(16):
                        wsp = jnp.full((16,), wvec[t], jnp.float32)
                        for q in range(_H // 16):
                            sl = pl.ds(q * 16, 16)
                            rows[j][g * 16 + t, sl] = rows[j][g * 16 + t, sl] * wsp
                    return 0

                lax.fori_loop(0, _C // 16, mul_body, 0)

                scatter(j).start(add=True)
            return 0

        lax.fori_loop(0, n_chunks // _NB, body, 0)
        scatter(_NB - 1).wait()

        plsc.subcore_barrier()

        @pl.when(s < nz_tiles)
        def _():
            pltpu.sync_copy(acc.at[pl.ds(row0, _BR)], out.at[c, pl.ds(row0, _BR)])

    return segsum


def _dot(a, b):
    return jnp.dot(a, b, preferred_element_type=jnp.float32)


def _k2_body(s0_ref, s1_ref, x_ref, h_ref, w0zr_a, w0zr_b, bzr_ref, wzr_a,
             wzr_b, w0c_a, w0c_b, w1c_a, bc_ref, z_out, rh_out, dp_out):
    x = x_ref[:, :]
    h = h_ref[:, :]
    dzr = _dot(x, w0zr_a[:, :]) + _dot(h, w0zr_b[:, :]) + bzr_ref[:, :]
    s0 = s0_ref[:, :]
    szr = _dot(s0, wzr_a[:, :]) + _dot(s1_ref[:, :], wzr_b[:, :])
    zr = jax.nn.sigmoid(dzr + szr)
    z = zr[:, :_H]
    r = zr[:, _H:]
    rh = r * h
    z_out[:, :] = z
    rh_out[:, :] = rh
    dp_out[:, :] = (_dot(x, w0c_a[:, :]) + _dot(rh, w0c_b[:, :])
                    + bc_ref[:, :] + _dot(s0, w1c_a[:, :]))


def _k3_body(z_ref, h_ref, dp_ref, sr0_ref, sr1_ref, w1c_b, out_ref):
    srh = sr0_ref[:, :] + sr1_ref[:, :]
    cc = jnp.tanh(dp_ref[:, :] + _dot(srh, w1c_b[:, :]))
    z = z_ref[:, :]
    out_ref[:, :] = z * h_ref[:, :] + (1.0 - z) * cc


def _kout_body(h_ref, w_ref, b_ref, out_ref):
    out_ref[:, :] = _dot(h_ref[:, :], w_ref[:, :]) + b_ref[:, :]


def _row_spec(w):
    return pl.BlockSpec((_BN, w), lambda i: (i, 0))


def _full_spec(r, w):
    return pl.BlockSpec((r, w), lambda i: (0, 0))


def _k2(n, s0, s1, x, h, w0zr_a, w0zr_b, bzr, wzr_a, wzr_b, w0c_a, w0c_b,
        w1c_a, bc):
    return pl.pallas_call(
        _k2_body,
        grid=(n // _BN,),
        in_specs=[_row_spec(_H), _row_spec(_H), _row_spec(_H), _row_spec(_H),
                  _full_spec(_H, 2 * _H), _full_spec(_H, 2 * _H),
                  _full_spec(1, 2 * _H),
                  _full_spec(_H, 2 * _H), _full_spec(_H, 2 * _H),
                  _full_spec(_H, _H), _full_spec(_H, _H), _full_spec(_H, _H),
                  _full_spec(1, _H)],
        out_specs=[_row_spec(_H), _row_spec(_H), _row_spec(_H)],
        out_shape=[jax.ShapeDtypeStruct((n, _H), jnp.float32)] * 3,
    )(s0, s1, x, h, w0zr_a, w0zr_b, bzr, wzr_a, wzr_b, w0c_a, w0c_b, w1c_a, bc)


def _k3(n, z, h, dp, sr0, sr1, w1c_b):
    return pl.pallas_call(
        _k3_body,
        grid=(n // _BN,),
        in_specs=[_row_spec(_H), _row_spec(_H), _row_spec(_H), _row_spec(_H),
                  _row_spec(_H), _full_spec(_H, _H)],
        out_specs=_row_spec(_H),
        out_shape=jax.ShapeDtypeStruct((n, _H), jnp.float32),
    )(z, h, dp, sr0, sr1, w1c_b)


def _kout(n, h, w, b):
    return pl.pallas_call(
        _kout_body,
        grid=(n // _BN,),
        in_specs=[_row_spec(_H), _full_spec(_H, _H), _full_spec(1, _H)],
        out_specs=_row_spec(_H),
        out_shape=jax.ShapeDtypeStruct((n, _H), jnp.float32),
    )(h, w, b)


def kernel(x, edge_index, edge_weight, cells, W_out, b_out):
    n, f, t_steps = x.shape
    assert f == _H and n % _NS == 0 and n % _BN == 0

    src = edge_index[0]
    dst = edge_index[1]
    w = edge_weight.astype(jnp.float32)
    e = src.shape[0]

    epad = -(-e // (_NC * _NS * _C)) * (_NC * _NS * _C)
    if epad != e:
        pad = epad - e
        src = jnp.concatenate([src, jnp.zeros((pad,), jnp.int32)])
        dst = jnp.concatenate([dst, jnp.zeros((pad,), jnp.int32)])
        w = jnp.concatenate([w, jnp.zeros((pad,), jnp.float32)])

    seg_pg = _make_segsum(n, epad, False)
    seg_sp = _make_segsum(n, epad, True)

    prep = []
    for cell in cells:
        w0zr = jnp.concatenate([cell["W0_z"], cell["W0_r"]], axis=1)
        w1zr = jnp.concatenate([cell["W1_z"], cell["W1_r"]], axis=1)
        prep.append(dict(
            w0zr_a=w0zr[:_H], w0zr_b=w0zr[_H:],
            wzr_a=w1zr[:_H], wzr_b=w1zr[_H:],
            w0c_a=cell["W0_c"][:_H], w0c_b=cell["W0_c"][_H:],
            w1c_a=cell["W1_c"][:_H], w1c_b=cell["W1_c"][_H:],
            bzr=jnp.concatenate([cell["b_z"], cell["b_r"]])[None, :],
            bc=cell["b_c"][None, :],
        ))

    x_t_major = jnp.transpose(x, (2, 0, 1))
    h_state = [jnp.zeros((n, _H), jnp.float32) for _ in range(len(cells))]

    for t in range(t_steps):
        xin = x_t_major[t]
        for l, p in enumerate(prep):
            h_prev = h_state[l]
            s_agg = seg_pg(xin, h_prev, src, dst, w)
            z, rh, dp = _k2(n, s_agg[0], s_agg[1], xin, h_prev,
                            p["w0zr_a"], p["w0zr_b"], p["bzr"],
                            p["wzr_a"], p["wzr_b"], p["w0c_a"], p["w0c_b"],
                            p["w1c_a"], p["bc"])
            srh = seg_sp(rh, rh, src, dst, w)
            h_new = _k3(n, z, h_prev, dp, srh[0], srh[1], p["w1c_b"])
            h_state[l] = h_new
            xin = h_new

    return _kout(n, h_state[-1], W_out, b_out[None, :])


# speedup vs baseline: 1.0479x; 1.0479x over previous
import functools

import jax
import jax.numpy as jnp
from jax import lax
from jax.experimental import pallas as pl
from jax.experimental.pallas import tpu as pltpu
from jax.experimental.pallas import tpu_sc as plsc

_C = 64
_NS = 16
_NC = 2
_H = 128
_BN = 1000


_NB = 4


@functools.lru_cache(maxsize=None)
def _make_segsum(n_nodes, n_edges, split):
    per_tile = n_edges // (_NS * (_NC if split else 1))
    n_chunks = per_tile // _C
    assert per_tile % _C == 0 and n_chunks % _NB == 0
    _BR = 1000
    nz_tiles = n_nodes // _BR

    mesh = plsc.VectorSubcoreMesh(core_axis_name="c", subcore_axis_name="s")

    scratch = (
        [pltpu.VMEM((per_tile,), jnp.int32)]
        + [pltpu.VMEM((_C, _H), jnp.float32) for _ in range(_NB)]
        + [pltpu.VMEM((_C,), jnp.int32) for _ in range(_NB)]
        + [pltpu.VMEM((_C,), jnp.float32) for _ in range(_NB)]
        + [pltpu.VMEM_SHARED((n_nodes, _H), jnp.float32)]
        + [pltpu.SemaphoreType.DMA] * (3 * _NB + 1)
    )

    @functools.partial(
        pl.kernel,
        mesh=mesh,
        out_type=jax.ShapeDtypeStruct((_NC, n_nodes, _H), jnp.float32),
        scratch_types=scratch,
    )
    def segsum(tab_a, tab_b, srcv, dstv, wv, out, src_all, *bufs):
        rows = bufs[:_NB]
        dst_v = bufs[_NB:2 * _NB]
        w_v = bufs[2 * _NB:3 * _NB]
        acc = bufs[3 * _NB]
        gsem = bufs[3 * _NB + 1:4 * _NB + 1]
        ssem = bufs[4 * _NB + 1:5 * _NB + 1]
        dsem = bufs[5 * _NB + 1:6 * _NB + 1]
        isem = bufs[6 * _NB + 1]

        c = lax.axis_index("c")
        s = lax.axis_index("s")

        if split:
            base_e = (c * _NS + s) * per_tile
        else:
            base_e = s * per_tile

        cp_s = pltpu.async_copy(srcv.at[pl.ds(base_e, per_tile)], src_all, isem)

        def zero_body(i, _):
            for j in range(_H // 16):
                rows[0][i, pl.ds(j * 16, 16)] = jnp.zeros((16,), jnp.float32)
            return 0

        lax.fori_loop(0, _C, zero_body, 0)
        row0 = s * _BR

        @pl.when(s < nz_tiles)
        def _():
            off = 0
            while off < _BR:
                nn = min(_C, _BR - off)
                pltpu.sync_copy(rows[0].at[pl.ds(0, nn)], acc.at[pl.ds(row0 + off, nn)])
                off += nn

        cp_s.wait()

        _NSUB = 8
        _SCH = _C // _NSUB

        def _gather_parts(tab, k, j):
            return [pltpu.make_async_copy(
                        tab.at[src_all.at[pl.ds(k * _C + i * _SCH, _SCH)]],
                        rows[j].at[pl.ds(i * _SCH, _SCH)], gsem[j])
                    for i in range(_NSUB)]

        def gather_start(k, j):
            if split:
                for cp in _gather_parts(tab_a, k, j):
                    cp.start()
            else:
                @pl.when(c == 0)
                def _():
                    for cp in _gather_parts(tab_a, k, j):
                        cp.start()

                @pl.when(c == 1)
                def _():
                    for cp in _gather_parts(tab_b, k, j):
                        cp.start()

        def gather_wait(k, j):
            for cp in _gather_parts(tab_a, k, j):
                cp.wait()

        def dstcp(k, j):
            return pltpu.make_async_copy(
                dstv.at[pl.ds(base_e + k * _C, _C)], dst_v[j], dsem[j])

        def wcp(k, j):
            return pltpu.make_async_copy(
                wv.at[pl.ds(base_e + k * _C, _C)], w_v[j], dsem[j])

        def scatter(j):
            return pltpu.make_async_copy(rows[j], acc.at[dst_v[j]], ssem[j])

        for j in range(_NB - 1):
            gather_start(j, j)
            dstcp(j, j).start()
            wcp(j, j).start()

        plsc.subcore_barrier()

        def body(kk, _):
            for j in range(_NB):
                k = kk * _NB + j
                gather_wait(k, j)

                dstcp(k, j).wait()
                wcp(k, j).wait()

                pb = (j + _NB - 1) % _NB
                pf = k + _NB - 1
                prev = k - 1

                @pl.when(prev >= 0)
                def _():
                    scatter(pb).wait()

                @pl.when(pf < n_chunks)
                def _():
                    gather_start(pf, pb)
                    dstcp(pf, pb).start()
                    wcp(pf, pb).start()

                def mul_body(g, _):
                    wvec = w_v[j][pl.ds(g * 16, 16)]
                    for t in range(16):
                        wsp = jnp.full((16,), wvec[t], jnp.float32)
                        for q in range(_H // 16):
                            sl = pl.ds(q * 16, 16)
                            rows[j][g * 16 + t, sl] = rows[j][g * 16 + t, sl] * wsp
                    return 0

                lax.fori_loop(0, _C // 16, mul_body, 0)

                scatter(j).start(add=True)
            return 0

        lax.fori_loop(0, n_chunks // _NB, body, 0)
        scatter(_NB - 1).wait()

        plsc.subcore_barrier()

        @pl.when(s < nz_tiles)
        def _():
            pltpu.sync_copy(acc.at[pl.ds(row0, _BR)], out.at[c, pl.ds(row0, _BR)])

    return segsum


def _dot(a, b):
    return jnp.dot(a, b, preferred_element_type=jnp.float32)


def _k2_body(s0_ref, s1_ref, x_ref, h_ref, w0zr_a, w0zr_b, bzr_ref, wzr_a,
             wzr_b, w0c_a, w0c_b, w1c_a, bc_ref, z_out, rh_out, dp_out):
    x = x_ref[:, :]
    h = h_ref[:, :]
    dzr = _dot(x, w0zr_a[:, :]) + _dot(h, w0zr_b[:, :]) + bzr_ref[:, :]
    s0 = s0_ref[:, :]
    szr = _dot(s0, wzr_a[:, :]) + _dot(s1_ref[:, :], wzr_b[:, :])
    zr = jax.nn.sigmoid(dzr + szr)
    z = zr[:, :_H]
    r = zr[:, _H:]
    rh = r * h
    z_out[:, :] = z
    rh_out[:, :] = rh
    dp_out[:, :] = (_dot(x, w0c_a[:, :]) + _dot(rh, w0c_b[:, :])
                    + bc_ref[:, :] + _dot(s0, w1c_a[:, :]))


def _k3_body(z_ref, h_ref, dp_ref, sr0_ref, sr1_ref, w1c_b, out_ref):
    srh = sr0_ref[:, :] + sr1_ref[:, :]
    cc = jnp.tanh(dp_ref[:, :] + _dot(srh, w1c_b[:, :]))
    z = z_ref[:, :]
    out_ref[:, :] = z * h_ref[:, :] + (1.0 - z) * cc


def _kout_body(h_ref, w_ref, b_ref, out_ref):
    out_ref[:, :] = _dot(h_ref[:, :], w_ref[:, :]) + b_ref[:, :]


def _row_spec(w):
    return pl.BlockSpec((_BN, w), lambda i: (i, 0))


def _full_spec(r, w):
    return pl.BlockSpec((r, w), lambda i: (0, 0))


def _k2(n, s0, s1, x, h, w0zr_a, w0zr_b, bzr, wzr_a, wzr_b, w0c_a, w0c_b,
        w1c_a, bc):
    return pl.pallas_call(
        _k2_body,
        grid=(n // _BN,),
        in_specs=[_row_spec(_H), _row_spec(_H), _row_spec(_H), _row_spec(_H),
                  _full_spec(_H, 2 * _H), _full_spec(_H, 2 * _H),
                  _full_spec(1, 2 * _H),
                  _full_spec(_H, 2 * _H), _full_spec(_H, 2 * _H),
                  _full_spec(_H, _H), _full_spec(_H, _H), _full_spec(_H, _H),
                  _full_spec(1, _H)],
        out_specs=[_row_spec(_H), _row_spec(_H), _row_spec(_H)],
        out_shape=[jax.ShapeDtypeStruct((n, _H), jnp.float32)] * 3,
    )(s0, s1, x, h, w0zr_a, w0zr_b, bzr, wzr_a, wzr_b, w0c_a, w0c_b, w1c_a, bc)


def _k3(n, z, h, dp, sr0, sr1, w1c_b):
    return pl.pallas_call(
        _k3_body,
        grid=(n // _BN,),
        in_specs=[_row_spec(_H), _row_spec(_H), _row_spec(_H), _row_spec(_H),
                  _row_spec(_H), _full_spec(_H, _H)],
        out_specs=_row_spec(_H),
        out_shape=jax.ShapeDtypeStruct((n, _H), jnp.float32),
    )(z, h, dp, sr0, sr1, w1c_b)


def _kout(n, h, w, b):
    return pl.pallas_call(
        _kout_body,
        grid=(n // _BN,),
        in_specs=[_row_spec(_H), _full_spec(_H, _H), _full_spec(1, _H)],
        out_specs=_row_spec(_H),
        out_shape=jax.ShapeDtypeStruct((n, _H), jnp.float32),
    )(h, w, b)


def kernel(x, edge_index, edge_weight, cells, W_out, b_out):
    n, f, t_steps = x.shape
    assert f == _H and n % _NS == 0 and n % _BN == 0

    src = edge_index[0]
    dst = edge_index[1]
    w = edge_weight.astype(jnp.float32)
    e = src.shape[0]

    _G = _NC * _NS * _C * _NB
    epad = -(-e // _G) * _G
    if epad != e:
        pad = epad - e
        src = jnp.concatenate([src, jnp.zeros((pad,), jnp.int32)])
        dst = jnp.concatenate([dst, jnp.zeros((pad,), jnp.int32)])
        w = jnp.concatenate([w, jnp.zeros((pad,), jnp.float32)])

    seg_pg = _make_segsum(n, epad, False)
    seg_sp = _make_segsum(n, epad, True)

    prep = []
    for cell in cells:
        w0zr = jnp.concatenate([cell["W0_z"], cell["W0_r"]], axis=1)
        w1zr = jnp.concatenate([cell["W1_z"], cell["W1_r"]], axis=1)
        prep.append(dict(
            w0zr_a=w0zr[:_H], w0zr_b=w0zr[_H:],
            wzr_a=w1zr[:_H], wzr_b=w1zr[_H:],
            w0c_a=cell["W0_c"][:_H], w0c_b=cell["W0_c"][_H:],
            w1c_a=cell["W1_c"][:_H], w1c_b=cell["W1_c"][_H:],
            bzr=jnp.concatenate([cell["b_z"], cell["b_r"]])[None, :],
            bc=cell["b_c"][None, :],
        ))

    x_t_major = jnp.transpose(x, (2, 0, 1))
    h_state = [jnp.zeros((n, _H), jnp.float32) for _ in range(len(cells))]

    for t in range(t_steps):
        xin = x_t_major[t]
        for l, p in enumerate(prep):
            h_prev = h_state[l]
            s_agg = seg_pg(xin, h_prev, src, dst, w)
            z, rh, dp = _k2(n, s_agg[0], s_agg[1], xin, h_prev,
                            p["w0zr_a"], p["w0zr_b"], p["bzr"],
                            p["wzr_a"], p["wzr_b"], p["w0c_a"], p["w0c_b"],
                            p["w1c_a"], p["bc"])
            srh = seg_sp(rh, rh, src, dst, w)
            h_new = _k3(n, z, h_prev, dp, srh[0], srh[1], p["w1c_b"])
            h_state[l] = h_new
            xin = h_new

    return _kout(n, h_state[-1], W_out, b_out[None, :])
